# baseline (speedup 1.0000x reference)
import functools

import jax
import jax.numpy as jnp
from jax import lax
from jax.experimental import pallas as pl
from jax.experimental.pallas import tpu as pltpu
from jax.experimental.pallas import tpu_sc as plsc

_HB = 8
_NCH = 8


@functools.cache
def _gather_fn(V, D, B, H):
  info = plsc.get_sparse_core_info()
  NC, NS, L = info.num_cores, info.num_subcores, info.num_lanes
  NW = NC * NS
  BLK = B // NW
  G = H // _HB
  D8 = D // 8
  assert BLK == 128 and H % _HB == 0 and G >= 4 and D % 8 == 0 and L == 16
  assert BLK % _NCH == 0
  RCH = BLK // _NCH
  mesh = plsc.VectorSubcoreMesh(core_axis_name="c", subcore_axis_name="s")

  @functools.partial(
      pl.kernel, mesh=mesh,
      out_type=jax.ShapeDtypeStruct((H, D8, NW, 8, 128), jnp.float32),
      compiler_params=pltpu.CompilerParams(
          use_tc_tiling_on_sc=False, needs_layout_passes=False),
      scratch_types=[
          pltpu.VMEM((2, _HB, BLK), jnp.int32),
          pltpu.VMEM((2, _HB, BLK, D), jnp.float32),
          pltpu.VMEM((_HB, D8, 8, 129), jnp.float32),
          pltpu.SemaphoreType.DMA,
          pltpu.SemaphoreType.DMA,
          pltpu.SemaphoreType.DMA,
          pltpu.SemaphoreType.DMA,
          pltpu.SemaphoreType.DMA,
      ],
  )
  def k(xt_hbm, table_hbm, out_hbm, idx_v, rows_v, tiles_v,
        isem0, isem1, gsem0, gsem1, wsem):
    isems, gsems = (isem0, isem1), (gsem0, gsem1)
    wid = lax.axis_index("s") * NC + lax.axis_index("c")
    bb0 = wid * BLK

    lanes = lax.iota(jnp.int32, L)
    batch_lanes = [lanes + c * L for c in range(BLK // L)]

    def idx_start(h, b):
      g = jnp.where(h < G, h, 0)
      pltpu.async_copy(xt_hbm.at[g, wid], idx_v.at[b], isems[b])

    def idx_wait(b):
      pltpu.make_async_copy(xt_hbm.at[0, wid], idx_v.at[b], isems[b]).wait()

    def fire(b):
      for h in range(_HB):
        pltpu.async_copy(
            table_hbm.at[idx_v.at[b, h]], rows_v.at[b, h], gsems[b])

    def drain(b):
      for h in range(_HB):
        pltpu.make_async_copy(
            table_hbm.at[idx_v.at[b, h]], rows_v.at[b, h], gsems[b]).wait()

    _U = 16

    def transpose(b):
      zeros = jnp.zeros((L,), jnp.int32)
      d8lo, drlo = lanes // 8, lanes % 8
      d8hi = d8lo + (L // 8)

      @pl.loop(0, _HB * (BLK // _U))
      def _(q):
        h, r0 = q // (BLK // _U), (q % (BLK // _U)) * _U
        hvec = zeros + h
        vs = []
        for u in range(_U):
          vs.append((rows_v[b, h, r0 + u, pl.ds(0, L)],
                     rows_v[b, h, r0 + u, pl.ds(L, L)]))
        for u, (v0, v1) in enumerate(vs):
          cvec = zeros + (r0 + u)
          plsc.store_scatter(tiles_v, [hvec, d8lo, drlo, cvec], v0)
          plsc.store_scatter(tiles_v, [hvec, d8hi, drlo, cvec], v1)

    def wb_start(g):
      pltpu.async_copy(
          tiles_v.at[:, :, :, pl.ds(0, 128)],
          out_hbm.at[pl.ds(g * _HB, _HB), :, wid], wsem)

    def wb_wait():
      pltpu.make_async_copy(
          tiles_v.at[:, :, :, pl.ds(0, 128)],
          out_hbm.at[pl.ds(0, _HB), :, wid], wsem).wait()

    idx_start(0, 0)
    idx_wait(0)
    fire(0)
    idx_start(1, 1)
    drain(0)
    idx_start(2, 0)
    idx_wait(1)
    fire(1)
    transpose(0)
    wb_start(0)

    @pl.loop(0, (G - 3) // 2)
    def _(i):
      for g_off, b in ((0, 1), (1, 0)):
        g = 1 + 2 * i + g_off
        ob = 1 - b
        drain(b)
        idx_start(g + 2, b)
        idx_wait(ob)
        fire(ob)
        wb_wait()
        transpose(b)
        wb_start(g)

    drain(1)
    idx_start(G, 1)
    idx_wait(0)
    fire(0)
    wb_wait()
    transpose(1)
    wb_start(G - 2)

    drain(0)
    idx_wait(1)
    wb_wait()
    transpose(0)
    wb_start(G - 1)
    wb_wait()

  return k


def kernel(x, table):
  B, H = x.shape
  V, D = table.shape
  xt = x.T.reshape(H // 8, 8, B // 128, 128).transpose(0, 2, 1, 3)
  p = _gather_fn(V, D, B, H)(xt, table)
  return p.transpose(2, 4, 0, 1, 3).reshape(B, H, D)


# speedup vs baseline: 1.0213x; 1.0213x over previous
import functools

import jax
import jax.numpy as jnp
from jax import lax
from jax.experimental import pallas as pl
from jax.experimental.pallas import tpu as pltpu
from jax.experimental.pallas import tpu_sc as plsc

_HB = 8
_NCH = 8


@functools.cache
def _gather_fn(V, D, B, H):
  info = plsc.get_sparse_core_info()
  NC, NS, L = info.num_cores, info.num_subcores, info.num_lanes
  NW = NC * NS
  BLK = B // NW
  G = H // _HB
  D8 = D // 8
  assert BLK == 128 and H % _HB == 0 and G >= 4 and D % 8 == 0 and L == 16
  assert BLK % _NCH == 0
  RCH = BLK // _NCH
  mesh = plsc.VectorSubcoreMesh(core_axis_name="c", subcore_axis_name="s")

  @functools.partial(
      pl.kernel, mesh=mesh,
      out_type=jax.ShapeDtypeStruct((H, D8, NW, 8, 128), jnp.float32),
      compiler_params=pltpu.CompilerParams(
          use_tc_tiling_on_sc=False, needs_layout_passes=False),
      scratch_types=[
          pltpu.VMEM((2, _HB, BLK), jnp.int32),
          pltpu.VMEM((2, _HB, BLK, D), jnp.float32),
          pltpu.VMEM((_HB, D8, 8, 129), jnp.float32),
          pltpu.SemaphoreType.DMA,
          pltpu.SemaphoreType.DMA,
          pltpu.SemaphoreType.DMA,
          pltpu.SemaphoreType.DMA,
          pltpu.SemaphoreType.DMA,
      ],
  )
  def k(xt_hbm, table_hbm, out_hbm, idx_v, rows_v, tiles_v,
        isem0, isem1, gsem0, gsem1, wsem):
    isems, gsems = (isem0, isem1), (gsem0, gsem1)
    wid = lax.axis_index("s") * NC + lax.axis_index("c")
    bb0 = wid * BLK

    lanes = lax.iota(jnp.int32, L)
    batch_lanes = [lanes + c * L for c in range(BLK // L)]

    def idx_start(h, b):
      g = jnp.where(h < G, h, 0)
      pltpu.async_copy(xt_hbm.at[g, wid], idx_v.at[b], isems[b])

    def idx_wait(b):
      pltpu.make_async_copy(xt_hbm.at[0, wid], idx_v.at[b], isems[b]).wait()

    def fire(b):
      for h in range(_HB):
        pltpu.async_copy(
            table_hbm.at[idx_v.at[b, h]], rows_v.at[b, h], gsems[b])

    def drain(b):
      for h in range(_HB):
        pltpu.make_async_copy(
            table_hbm.at[idx_v.at[b, h]], rows_v.at[b, h], gsems[b]).wait()

    _U = 8

    def transpose(b):
      zeros = jnp.zeros((L,), jnp.int32)
      d8lo, drlo = lanes // 8, lanes % 8
      d8hi = d8lo + (L // 8)

      @pl.loop(0, _HB * (BLK // _U))
      def _(q):
        h, r0 = q // (BLK // _U), (q % (BLK // _U)) * _U
        hvec = zeros + h
        vs = []
        for u in range(_U):
          vs.append((rows_v[b, h, r0 + u, pl.ds(0, L)],
                     rows_v[b, h, r0 + u, pl.ds(L, L)]))
        for u, (v0, v1) in enumerate(vs):
          cvec = zeros + (r0 + u)
          plsc.store_scatter(tiles_v, [hvec, d8lo, drlo, cvec], v0)
          plsc.store_scatter(tiles_v, [hvec, d8hi, drlo, cvec], v1)

    def wb_start(g):
      pltpu.async_copy(
          tiles_v.at[:, :, :, pl.ds(0, 128)],
          out_hbm.at[pl.ds(g * _HB, _HB), :, wid], wsem)

    def wb_wait():
      pltpu.make_async_copy(
          tiles_v.at[:, :, :, pl.ds(0, 128)],
          out_hbm.at[pl.ds(0, _HB), :, wid], wsem).wait()

    idx_start(0, 0)
    idx_wait(0)
    fire(0)
    idx_start(1, 1)
    drain(0)
    idx_start(2, 0)
    idx_wait(1)
    fire(1)
    transpose(0)
    wb_start(0)

    @pl.loop(0, (G - 3) // 2)
    def _(i):
      for g_off, b in ((0, 1), (1, 0)):
        g = 1 + 2 * i + g_off
        ob = 1 - b
        idx_wait(ob)
        fire(ob)
        drain(b)
        idx_start(g + 2, b)
        wb_wait()
        transpose(b)
        wb_start(g)

    idx_wait(0)
    fire(0)
    drain(1)
    idx_start(G, 1)
    wb_wait()
    transpose(1)
    wb_start(G - 2)

    drain(0)
    idx_wait(1)
    wb_wait()
    transpose(0)
    wb_start(G - 1)
    wb_wait()

  return k


def kernel(x, table):
  B, H = x.shape
  V, D = table.shape
  xt = x.T.reshape(H // 8, 8, B // 128, 128).transpose(0, 2, 1, 3)
  p = _gather_fn(V, D, B, H)(xt, table)
  return p.transpose(2, 4, 0, 1, 3).reshape(B, H, D)
